# baseline (speedup 1.0000x reference)
import functools

import jax
import jax.numpy as jnp
from jax import lax
from jax.experimental import pallas as pl
from jax.experimental.pallas import tpu as pltpu
from jax.experimental.pallas import tpu_sc as plsc

CH = 128
NW, NS, L = 32, 16, 16


def _node_y2_body(h_ref, xp_ref, wx_ref, bx_ref, y2_ref):
    wv = lax.dot_general(h_ref[...], wx_ref[...], (((1,), (1,)), ((), ())),
                         preferred_element_type=jnp.float32)
    wv = jnp.maximum(wv + bx_ref[0, 0], 0.0)
    lane = lax.broadcasted_iota(jnp.int32, y2_ref.shape, 1)
    y2_ref[...] = xp_ref[...] * wv + jnp.where(lane == 3, wv, 0.0)


def _node_y2_stage(h, xp, W_x, b_x, bn):
    n, d = h.shape
    grid = (n // bn,)
    return pl.pallas_call(
        _node_y2_body,
        grid=grid,
        in_specs=[
            pl.BlockSpec((bn, d), lambda i: (i, 0)),
            pl.BlockSpec((bn, 4), lambda i: (i, 0)),
            pl.BlockSpec((4, d), lambda i: (0, 0)),
            pl.BlockSpec(memory_space=pltpu.SMEM),
        ],
        out_specs=pl.BlockSpec((bn, 4), lambda i: (i, 0)),
        out_shape=jax.ShapeDtypeStruct((n, 4), jnp.float32),
    )(h, xp, W_x, b_x)


def _node_y1_body(h_ref, wh_ref, bh_ref, y1_ref):
    m = lax.dot_general(h_ref[...], wh_ref[...], (((1,), (1,)), ((), ())),
                        preferred_element_type=jnp.float32)
    y1_ref[...] = jnp.maximum(m + bh_ref[...], 0.0)


def _node_y1_stage(h, W_h, b_h, bn):
    n, d = h.shape
    grid = (n // bn,)
    return pl.pallas_call(
        _node_y1_body,
        grid=grid,
        in_specs=[
            pl.BlockSpec((bn, d), lambda i: (i, 0)),
            pl.BlockSpec((d, d), lambda i: (0, 0)),
            pl.BlockSpec((1, d), lambda i: (0, 0)),
        ],
        out_specs=pl.BlockSpec((bn, d), lambda i: (i, 0)),
        out_shape=jax.ShapeDtypeStruct((n, d), jnp.float32),
    )(h, W_h, b_h)


Q0, Q1 = 132, 26


def _sc_wide_stage(y1, colf, rowf, n_acc, nq):
    n, d = y1.shape
    r = n_acc // NS
    qmax = max(Q0, Q1)
    mesh = plsc.VectorSubcoreMesh(core_axis_name="c", subcore_axis_name="s")

    @functools.partial(
        pl.kernel,
        mesh=mesh,
        out_type=jax.ShapeDtypeStruct((2, n_acc, d), jnp.float32),
        scratch_types=[
            pltpu.VMEM((2, CH), jnp.int32),
            pltpu.VMEM((2, CH), jnp.int32),
            pltpu.VMEM((2, CH, d), jnp.float32),
            pltpu.VMEM_SHARED((n_acc, d), jnp.float32),
            pltpu.SemaphoreType.DMA((2,)),
            pltpu.SemaphoreType.DMA((2,)),
            pltpu.SemaphoreType.DMA((2,)),
        ],
    )
    def sc_wide(y1_hbm, col_hbm, row_hbm, p1_hbm,
                cvb, rvb, rows1_v, acc1, csems, rsems, gsems):
        c = lax.axis_index("c")
        s = lax.axis_index("s")
        r0 = s * r
        my_q = jnp.where(c == 0, Q0, Q1)
        base = jnp.where(c == 0, s * Q0, NS * Q0 + s * Q1)

        def zstep(tk, carry):
            rows1_v[0, tk // 8, pl.ds((tk % 8) * L, L)] = jnp.zeros(
                (L,), jnp.float32)
            return carry

        lax.fori_loop(0, CH * 8, zstep, 0)
        nz = r // CH
        for zi in range(nz):
            pltpu.sync_copy(rows1_v.at[0], acc1.at[pl.ds(r0 + zi * CH, CH)])
        if r % CH:
            pltpu.sync_copy(rows1_v.at[0, pl.ds(0, r % CH)],
                            acc1.at[pl.ds(r0 + nz * CH, r % CH)])
        plsc.subcore_barrier()

        def fetch_col(j, t):
            pltpu.async_copy(col_hbm.at[base + j], cvb.at[t], csems.at[t])

        def fetch_row(j, t):
            pltpu.async_copy(row_hbm.at[base + j], rvb.at[t], rsems.at[t])

        def wait_col(j, t):
            pltpu.make_async_copy(col_hbm.at[base + j], cvb.at[t],
                                  csems.at[t]).wait()

        def wait_row(j, t):
            pltpu.make_async_copy(row_hbm.at[base + j], rvb.at[t],
                                  rsems.at[t]).wait()

        def start_gather(t):
            pltpu.async_copy(y1_hbm.at[cvb.at[t]], rows1_v.at[t], gsems.at[t])

        def wait_gather(t):
            pltpu.make_async_copy(y1_hbm.at[cvb.at[t]], rows1_v.at[t],
                                  gsems.at[t]).wait()

        @pl.when(0 < my_q)
        def _():
            fetch_col(0, 0)
            fetch_row(0, 0)

        @pl.when(1 < my_q)
        def _():
            fetch_col(1, 1)
            fetch_row(1, 1)

        @pl.when(0 < my_q)
        def _():
            wait_col(0, 0)
            start_gather(0)

        def step(j, carry):
            t = j % 2

            @pl.when(j + 1 < my_q)
            def _():
                wait_col(j + 1, 1 - t)
                start_gather(1 - t)

            @pl.when(j < my_q)
            def _():
                wait_gather(t)

            @pl.when(j + 2 < my_q)
            def _():
                fetch_col(j + 2, t)

            @pl.when(j < my_q)
            def _():
                wait_row(j, t)
                pltpu.sync_copy(rows1_v.at[t], acc1.at[rvb.at[t]], add=True)

            @pl.when(j + 2 < my_q)
            def _():
                fetch_row(j + 2, t)
            return carry

        lax.fori_loop(0, qmax, step, 0)
        plsc.subcore_barrier()
        pltpu.sync_copy(acc1.at[pl.ds(r0, r)], p1_hbm.at[c, pl.ds(r0, r)])

    return sc_wide(y1, colf, rowf)


def _sc_narrow_stage(y2f, col3, row3, n_acc, nch, n):
    mesh = plsc.VectorSubcoreMesh(core_axis_name="c", subcore_axis_name="s")

    @functools.partial(
        pl.kernel,
        mesh=mesh,
        compiler_params=pltpu.CompilerParams(needs_layout_passes=False),
        out_type=jax.ShapeDtypeStruct((NW, n_acc * 4), jnp.float32),
        scratch_types=[
            pltpu.VMEM((nch, CH), jnp.int32),
            pltpu.VMEM((nch, CH), jnp.int32),
            pltpu.VMEM((n * 4,), jnp.float32),
            pltpu.VMEM((n_acc * 4,), jnp.float32),
        ],
    )
    def sc_narrow(y2_hbm, col_hbm, row_hbm, p2_hbm,
                  col_v, row_v, y2_v, acc2_v):
        c = lax.axis_index("c")
        s = lax.axis_index("s")
        b = c * NS + s

        def zstep(tk, carry):
            for u in range(8):
                acc2_v[pl.ds((tk * 8 + u) * L, L)] = jnp.zeros(
                    (L,), jnp.float32)
            return carry

        lax.fori_loop(0, n_acc * 4 // (8 * L), zstep, 0)
        pltpu.sync_copy(col_hbm.at[b], col_v)
        pltpu.sync_copy(row_hbm.at[b], row_v)
        pltpu.sync_copy(y2_hbm, y2_v)

        def step2(j, carry):
            for k in range(CH // L):
                c16 = col_v[j, pl.ds(k * L, L)]
                r16 = row_v[j, pl.ds(k * L, L)]
                for k4 in range(4):
                    vals = plsc.load_gather(y2_v, [c16 + k4])
                    plsc.addupdate_scatter(acc2_v, [r16 + k4 * n_acc], vals)
            return carry

        lax.fori_loop(0, nch, step2, 0)
        pltpu.sync_copy(acc2_v, p2_hbm.at[b])

    return sc_narrow(y2f, col3, row3)


def _combine_body(h_ref, xpt_ref, p1_ref, p2_ref, hn_ref, xnt_ref):
    s1 = p1_ref[0] + p1_ref[1]
    hn_ref[...] = h_ref[...] + s1

    @pl.when(pl.program_id(0) == 0)
    def _():
        s2t = jnp.sum(p2_ref[...], axis=0)
        wsum = s2t[3:4, :]
        krow = lax.broadcasted_iota(jnp.int32, s2t.shape, 0)
        xw = jnp.where(krow < 3, s2t, 0.0)
        xptb = xpt_ref[...]
        xnt_ref[...] = xptb + xptb * wsum - xw


def _combine_stage(h, xpt, p1, p2, bn):
    n, d = h.shape
    na = xpt.shape[1]
    grid = (n // bn,)
    return pl.pallas_call(
        _combine_body,
        grid=grid,
        in_specs=[
            pl.BlockSpec((bn, d), lambda i: (i, 0)),
            pl.BlockSpec((4, na), lambda i: (0, 0)),
            pl.BlockSpec((2, bn, d), lambda i: (0, i, 0)),
            pl.BlockSpec((NW, 4, na), lambda i: (0, 0, 0)),
        ],
        out_specs=[
            pl.BlockSpec((bn, d), lambda i: (i, 0)),
            pl.BlockSpec((4, na), lambda i: (0, 0)),
        ],
        out_shape=[
            jax.ShapeDtypeStruct((n, d), jnp.float32),
            jax.ShapeDtypeStruct((4, na), jnp.float32),
        ],
    )(h, xpt, p1, p2)


def kernel(h, x, edge_index, W_h, b_h, W_x, b_x):
    n, d = h.shape
    e = edge_index.shape[1]
    nch = -(-e // (NW * CH))
    e_pad = NW * nch * CH
    n_acc = (-(-(n + 1) // (NS * 8)) * 8) * NS

    row = edge_index[0].astype(jnp.int32)
    col = edge_index[1].astype(jnp.int32)
    colp = jnp.concatenate([col, jnp.zeros((e_pad - e,), jnp.int32)])
    rowp = jnp.concatenate([row, jnp.full((e_pad - e,), n, jnp.int32)])
    col3 = colp.reshape(NW, nch, CH)
    row3 = rowp.reshape(NW, nch, CH)
    xp = jnp.concatenate([x, jnp.zeros((n, 1), jnp.float32)], axis=1)

    wx4 = jnp.broadcast_to(W_x, (4, d))
    y2 = _node_y2_stage(h, xp, wx4, b_x.reshape(1, 1), bn=1000)
    nq = e_pad // CH
    p2 = _sc_narrow_stage(y2.reshape(-1), (col3 * 4), row3, n_acc, nch, n)
    y1 = _node_y1_stage(h, W_h, b_h.reshape(1, d), bn=1000)
    p1 = _sc_wide_stage(y1, colp.reshape(nq, CH), rowp.reshape(nq, CH),
                        n_acc, nq)
    xpt = jnp.concatenate(
        [x.T, jnp.zeros((1, n), jnp.float32)], axis=0)
    xpt = jnp.concatenate(
        [xpt, jnp.zeros((4, n_acc - n), jnp.float32)], axis=1)
    h_new, xnt = _combine_stage(h, xpt, p1, p2.reshape(NW, 4, n_acc), bn=1000)
    return (h_new, xnt[:3, :n].T)


# speedup vs baseline: 1.0629x; 1.0629x over previous
import functools

import jax
import jax.numpy as jnp
from jax import lax
from jax.experimental import pallas as pl
from jax.experimental.pallas import tpu as pltpu
from jax.experimental.pallas import tpu_sc as plsc

CH = 128
NW, NS, L = 32, 16, 16


def _node_body(h_ref, xp_ref, wh_ref, bh_ref, wx_ref, bx_ref, y1_ref, y2_ref):
    hb = h_ref[...]
    m = lax.dot_general(hb, wh_ref[...], (((1,), (1,)), ((), ())),
                        preferred_element_type=jnp.float32)
    y1_ref[...] = jnp.maximum(m + bh_ref[...], 0.0)
    wv = lax.dot_general(hb, wx_ref[...], (((1,), (1,)), ((), ())),
                         preferred_element_type=jnp.float32)
    wv = jnp.maximum(wv + bx_ref[0, 0], 0.0)
    lane = lax.broadcasted_iota(jnp.int32, y2_ref.shape, 1)
    y2_ref[...] = xp_ref[...] * wv + jnp.where(lane == 3, wv, 0.0)


def _node_stage(h, xp, W_h, b_h, W_x, b_x, bn):
    n, d = h.shape
    grid = (n // bn,)
    return pl.pallas_call(
        _node_body,
        grid=grid,
        in_specs=[
            pl.BlockSpec((bn, d), lambda i: (i, 0)),
            pl.BlockSpec((bn, 4), lambda i: (i, 0)),
            pl.BlockSpec((d, d), lambda i: (0, 0)),
            pl.BlockSpec((1, d), lambda i: (0, 0)),
            pl.BlockSpec((4, d), lambda i: (0, 0)),
            pl.BlockSpec(memory_space=pltpu.SMEM),
        ],
        out_specs=[
            pl.BlockSpec((bn, d), lambda i: (i, 0)),
            pl.BlockSpec((bn, 4), lambda i: (i, 0)),
        ],
        out_shape=[
            jax.ShapeDtypeStruct((n, d), jnp.float32),
            jax.ShapeDtypeStruct((n, 4), jnp.float32),
        ],
    )(h, xp, W_h, b_h, W_x, b_x)


Q0, Q1 = 132, 26


def _sc_wide_stage(y1, colf, rowf, n_acc, nq):
    n, d = y1.shape
    r = n_acc // NS
    qmax = max(Q0, Q1)
    mesh = plsc.VectorSubcoreMesh(core_axis_name="c", subcore_axis_name="s")

    @functools.partial(
        pl.kernel,
        mesh=mesh,
        out_type=jax.ShapeDtypeStruct((2, n_acc, d), jnp.float32),
        scratch_types=[
            pltpu.VMEM((2, CH), jnp.int32),
            pltpu.VMEM((2, CH), jnp.int32),
            pltpu.VMEM((2, CH, d), jnp.float32),
            pltpu.VMEM_SHARED((n_acc, d), jnp.float32),
            pltpu.SemaphoreType.DMA((2,)),
            pltpu.SemaphoreType.DMA((2,)),
            pltpu.SemaphoreType.DMA((2,)),
        ],
    )
    def sc_wide(y1_hbm, col_hbm, row_hbm, p1_hbm,
                cvb, rvb, rows1_v, acc1, csems, rsems, gsems):
        c = lax.axis_index("c")
        s = lax.axis_index("s")
        r0 = s * r
        my_q = jnp.where(c == 0, Q0, Q1)
        base = jnp.where(c == 0, s * Q0, NS * Q0 + s * Q1)

        def zstep(tk, carry):
            rows1_v[0, tk // 8, pl.ds((tk % 8) * L, L)] = jnp.zeros(
                (L,), jnp.float32)
            return carry

        lax.fori_loop(0, CH * 8, zstep, 0)
        nz = r // CH
        for zi in range(nz):
            pltpu.sync_copy(rows1_v.at[0], acc1.at[pl.ds(r0 + zi * CH, CH)])
        if r % CH:
            pltpu.sync_copy(rows1_v.at[0, pl.ds(0, r % CH)],
                            acc1.at[pl.ds(r0 + nz * CH, r % CH)])
        plsc.subcore_barrier()

        def fetch_col(j, t):
            pltpu.async_copy(col_hbm.at[base + j], cvb.at[t], csems.at[t])

        def fetch_row(j, t):
            pltpu.async_copy(row_hbm.at[base + j], rvb.at[t], rsems.at[t])

        def wait_col(j, t):
            pltpu.make_async_copy(col_hbm.at[base + j], cvb.at[t],
                                  csems.at[t]).wait()

        def wait_row(j, t):
            pltpu.make_async_copy(row_hbm.at[base + j], rvb.at[t],
                                  rsems.at[t]).wait()

        def start_gather(t):
            pltpu.async_copy(y1_hbm.at[cvb.at[t]], rows1_v.at[t], gsems.at[t])

        def wait_gather(t):
            pltpu.make_async_copy(y1_hbm.at[cvb.at[t]], rows1_v.at[t],
                                  gsems.at[t]).wait()

        @pl.when(0 < my_q)
        def _():
            fetch_col(0, 0)
            fetch_row(0, 0)

        @pl.when(1 < my_q)
        def _():
            fetch_col(1, 1)
            fetch_row(1, 1)

        @pl.when(0 < my_q)
        def _():
            wait_col(0, 0)
            start_gather(0)

        def step(j, carry):
            t = j % 2

            @pl.when(j + 1 < my_q)
            def _():
                wait_col(j + 1, 1 - t)
                start_gather(1 - t)

            @pl.when(j < my_q)
            def _():
                wait_gather(t)

            @pl.when(j + 2 < my_q)
            def _():
                fetch_col(j + 2, t)

            @pl.when(j < my_q)
            def _():
                wait_row(j, t)
                pltpu.sync_copy(rows1_v.at[t], acc1.at[rvb.at[t]], add=True)

            @pl.when(j + 2 < my_q)
            def _():
                fetch_row(j + 2, t)
            return carry

        lax.fori_loop(0, qmax, step, 0)
        plsc.subcore_barrier()
        pltpu.sync_copy(acc1.at[pl.ds(r0, r)], p1_hbm.at[c, pl.ds(r0, r)])

    return sc_wide(y1, colf, rowf)


def _sc_narrow_stage(y2f, col3, row3, n_acc, nch, n):
    mesh = plsc.VectorSubcoreMesh(core_axis_name="c", subcore_axis_name="s")

    @functools.partial(
        pl.kernel,
        mesh=mesh,
        compiler_params=pltpu.CompilerParams(needs_layout_passes=False),
        out_type=jax.ShapeDtypeStruct((NW, n_acc * 4), jnp.float32),
        scratch_types=[
            pltpu.VMEM((nch, CH), jnp.int32),
            pltpu.VMEM((nch, CH), jnp.int32),
            pltpu.VMEM((n * 4,), jnp.float32),
            pltpu.VMEM((n_acc * 4,), jnp.float32),
        ],
    )
    def sc_narrow(y2_hbm, col_hbm, row_hbm, p2_hbm,
                  col_v, row_v, y2_v, acc2_v):
        c = lax.axis_index("c")
        s = lax.axis_index("s")
        b = c * NS + s

        def zstep(tk, carry):
            for u in range(8):
                acc2_v[pl.ds((tk * 8 + u) * L, L)] = jnp.zeros(
                    (L,), jnp.float32)
            return carry

        lax.fori_loop(0, n_acc * 4 // (8 * L), zstep, 0)
        pltpu.sync_copy(col_hbm.at[b], col_v)
        pltpu.sync_copy(row_hbm.at[b], row_v)
        pltpu.sync_copy(y2_hbm, y2_v)

        def step2(j, carry):
            for k in range(CH // L):
                c16 = col_v[j, pl.ds(k * L, L)]
                r16 = row_v[j, pl.ds(k * L, L)]
                for k4 in range(4):
                    vals = plsc.load_gather(y2_v, [c16 + k4])
                    plsc.addupdate_scatter(acc2_v, [r16 + k4 * n_acc], vals)
            return carry

        lax.fori_loop(0, nch, step2, 0)
        pltpu.sync_copy(acc2_v, p2_hbm.at[b])

    return sc_narrow(y2f, col3, row3)


def _combine_body(h_ref, xpt_ref, p1_ref, p2_ref, hn_ref, xnt_ref):
    s1 = p1_ref[0] + p1_ref[1]
    hn_ref[...] = h_ref[...] + s1

    @pl.when(pl.program_id(0) == 0)
    def _():
        s2t = jnp.sum(p2_ref[...], axis=0)
        wsum = s2t[3:4, :]
        krow = lax.broadcasted_iota(jnp.int32, s2t.shape, 0)
        xw = jnp.where(krow < 3, s2t, 0.0)
        xptb = xpt_ref[...]
        xnt_ref[...] = xptb + xptb * wsum - xw


def _combine_stage(h, xpt, p1, p2, bn):
    n, d = h.shape
    na = xpt.shape[1]
    grid = (n // bn,)
    return pl.pallas_call(
        _combine_body,
        grid=grid,
        in_specs=[
            pl.BlockSpec((bn, d), lambda i: (i, 0)),
            pl.BlockSpec((4, na), lambda i: (0, 0)),
            pl.BlockSpec((2, bn, d), lambda i: (0, i, 0)),
            pl.BlockSpec((NW, 4, na), lambda i: (0, 0, 0)),
        ],
        out_specs=[
            pl.BlockSpec((bn, d), lambda i: (i, 0)),
            pl.BlockSpec((4, na), lambda i: (0, 0)),
        ],
        out_shape=[
            jax.ShapeDtypeStruct((n, d), jnp.float32),
            jax.ShapeDtypeStruct((4, na), jnp.float32),
        ],
    )(h, xpt, p1, p2)


def kernel(h, x, edge_index, W_h, b_h, W_x, b_x):
    n, d = h.shape
    e = edge_index.shape[1]
    nch = -(-e // (NW * CH))
    e_pad = NW * nch * CH
    n_acc = (-(-(n + 1) // (NS * 8)) * 8) * NS

    row = edge_index[0].astype(jnp.int32)
    col = edge_index[1].astype(jnp.int32)
    colp = jnp.concatenate([col, jnp.zeros((e_pad - e,), jnp.int32)])
    rowp = jnp.concatenate([row, jnp.full((e_pad - e,), n, jnp.int32)])
    col3 = colp.reshape(NW, nch, CH)
    row3 = rowp.reshape(NW, nch, CH)
    xp = jnp.concatenate([x, jnp.zeros((n, 1), jnp.float32)], axis=1)

    wx4 = jnp.broadcast_to(W_x, (4, d))
    y1, y2 = _node_stage(h, xp, W_h, b_h.reshape(1, d), wx4,
                         b_x.reshape(1, 1), bn=1000)
    nq = e_pad // CH
    p2 = _sc_narrow_stage(y2.reshape(-1), (col3 * 4), row3, n_acc, nch, n)
    p1 = _sc_wide_stage(y1, colp.reshape(nq, CH), rowp.reshape(nq, CH),
                        n_acc, nq)
    xpt = jnp.concatenate(
        [x.T, jnp.zeros((1, n), jnp.float32)], axis=0)
    xpt = jnp.concatenate(
        [xpt, jnp.zeros((4, n_acc - n), jnp.float32)], axis=1)
    h_new, xnt = _combine_stage(h, xpt, p1, p2.reshape(NW, 4, n_acc), bn=1000)
    return (h_new, xnt[:3, :n].T)


# speedup vs baseline: 1.0656x; 1.0026x over previous
import functools

import jax
import jax.numpy as jnp
from jax import lax
from jax.experimental import pallas as pl
from jax.experimental.pallas import tpu as pltpu
from jax.experimental.pallas import tpu_sc as plsc

CH = 128
NW, NS, L = 32, 16, 16


def _node_body(h_ref, xp_ref, wh_ref, bh_ref, wx_ref, bx_ref, y1_ref, y2_ref):
    hb = h_ref[...]
    m = lax.dot_general(hb, wh_ref[...], (((1,), (1,)), ((), ())),
                        preferred_element_type=jnp.float32)
    y1_ref[...] = jnp.maximum(m + bh_ref[...], 0.0)
    wv = lax.dot_general(hb, wx_ref[...], (((1,), (1,)), ((), ())),
                         preferred_element_type=jnp.float32)
    wv = jnp.maximum(wv + bx_ref[0, 0], 0.0)
    lane = lax.broadcasted_iota(jnp.int32, y2_ref.shape, 1)
    y2_ref[...] = xp_ref[...] * wv + jnp.where(lane == 3, wv, 0.0)


def _node_stage(h, xp, W_h, b_h, W_x, b_x, bn):
    n, d = h.shape
    grid = (n // bn,)
    return pl.pallas_call(
        _node_body,
        grid=grid,
        in_specs=[
            pl.BlockSpec((bn, d), lambda i: (i, 0)),
            pl.BlockSpec((bn, 4), lambda i: (i, 0)),
            pl.BlockSpec((d, d), lambda i: (0, 0)),
            pl.BlockSpec((1, d), lambda i: (0, 0)),
            pl.BlockSpec((4, d), lambda i: (0, 0)),
            pl.BlockSpec(memory_space=pltpu.SMEM),
        ],
        out_specs=[
            pl.BlockSpec((bn, d), lambda i: (i, 0)),
            pl.BlockSpec((bn, 4), lambda i: (i, 0)),
        ],
        out_shape=[
            jax.ShapeDtypeStruct((n, d), jnp.float32),
            jax.ShapeDtypeStruct((n, 4), jnp.float32),
        ],
    )(h, xp, W_h, b_h, W_x, b_x)


Q0, Q1 = 145, 13


def _sc_wide_stage(y1, colf, rowf, n_acc, nq):
    n, d = y1.shape
    r = n_acc // NS
    qmax = max(Q0, Q1)
    mesh = plsc.VectorSubcoreMesh(core_axis_name="c", subcore_axis_name="s")

    @functools.partial(
        pl.kernel,
        mesh=mesh,
        out_type=jax.ShapeDtypeStruct((2, n_acc, d), jnp.float32),
        scratch_types=[
            pltpu.VMEM((2, CH), jnp.int32),
            pltpu.VMEM((2, CH), jnp.int32),
            pltpu.VMEM((2, CH, d), jnp.float32),
            pltpu.VMEM_SHARED((n_acc, d), jnp.float32),
            pltpu.SemaphoreType.DMA((2,)),
            pltpu.SemaphoreType.DMA((2,)),
            pltpu.SemaphoreType.DMA((2,)),
        ],
    )
    def sc_wide(y1_hbm, col_hbm, row_hbm, p1_hbm,
                cvb, rvb, rows1_v, acc1, csems, rsems, gsems):
        c = lax.axis_index("c")
        s = lax.axis_index("s")
        r0 = s * r
        my_q = jnp.where(c == 0, Q0, Q1)
        base = jnp.where(c == 0, s * Q0, NS * Q0 + s * Q1)

        def zstep(tk, carry):
            rows1_v[0, tk // 8, pl.ds((tk % 8) * L, L)] = jnp.zeros(
                (L,), jnp.float32)
            return carry

        lax.fori_loop(0, CH * 8, zstep, 0)
        nz = r // CH
        for zi in range(nz):
            pltpu.sync_copy(rows1_v.at[0], acc1.at[pl.ds(r0 + zi * CH, CH)])
        if r % CH:
            pltpu.sync_copy(rows1_v.at[0, pl.ds(0, r % CH)],
                            acc1.at[pl.ds(r0 + nz * CH, r % CH)])
        plsc.subcore_barrier()

        def fetch_col(j, t):
            pltpu.async_copy(col_hbm.at[base + j], cvb.at[t], csems.at[t])

        def fetch_row(j, t):
            pltpu.async_copy(row_hbm.at[base + j], rvb.at[t], rsems.at[t])

        def wait_col(j, t):
            pltpu.make_async_copy(col_hbm.at[base + j], cvb.at[t],
                                  csems.at[t]).wait()

        def wait_row(j, t):
            pltpu.make_async_copy(row_hbm.at[base + j], rvb.at[t],
                                  rsems.at[t]).wait()

        def start_gather(t):
            pltpu.async_copy(y1_hbm.at[cvb.at[t]], rows1_v.at[t], gsems.at[t])

        def wait_gather(t):
            pltpu.make_async_copy(y1_hbm.at[cvb.at[t]], rows1_v.at[t],
                                  gsems.at[t]).wait()

        @pl.when(0 < my_q)
        def _():
            fetch_col(0, 0)
            fetch_row(0, 0)

        @pl.when(1 < my_q)
        def _():
            fetch_col(1, 1)
            fetch_row(1, 1)

        @pl.when(0 < my_q)
        def _():
            wait_col(0, 0)
            start_gather(0)

        def step(j, carry):
            t = j % 2

            @pl.when(j + 1 < my_q)
            def _():
                wait_col(j + 1, 1 - t)
                start_gather(1 - t)

            @pl.when(j < my_q)
            def _():
                wait_gather(t)

            @pl.when(j + 2 < my_q)
            def _():
                fetch_col(j + 2, t)

            @pl.when(j < my_q)
            def _():
                wait_row(j, t)
                pltpu.sync_copy(rows1_v.at[t], acc1.at[rvb.at[t]], add=True)

            @pl.when(j + 2 < my_q)
            def _():
                fetch_row(j + 2, t)
            return carry

        lax.fori_loop(0, qmax, step, 0)
        plsc.subcore_barrier()
        pltpu.sync_copy(acc1.at[pl.ds(r0, r)], p1_hbm.at[c, pl.ds(r0, r)])

    return sc_wide(y1, colf, rowf)


def _sc_narrow_stage(y2f, col3, row3, n_acc, nch, n):
    mesh = plsc.VectorSubcoreMesh(core_axis_name="c", subcore_axis_name="s")

    @functools.partial(
        pl.kernel,
        mesh=mesh,
        compiler_params=pltpu.CompilerParams(needs_layout_passes=False),
        out_type=jax.ShapeDtypeStruct((NW, n_acc * 4), jnp.float32),
        scratch_types=[
            pltpu.VMEM((nch, CH), jnp.int32),
            pltpu.VMEM((nch, CH), jnp.int32),
            pltpu.VMEM((n * 4,), jnp.float32),
            pltpu.VMEM((n_acc * 4,), jnp.float32),
        ],
    )
    def sc_narrow(y2_hbm, col_hbm, row_hbm, p2_hbm,
                  col_v, row_v, y2_v, acc2_v):
        c = lax.axis_index("c")
        s = lax.axis_index("s")
        b = c * NS + s

        def zstep(tk, carry):
            for u in range(8):
                acc2_v[pl.ds((tk * 8 + u) * L, L)] = jnp.zeros(
                    (L,), jnp.float32)
            return carry

        lax.fori_loop(0, n_acc * 4 // (8 * L), zstep, 0)
        pltpu.sync_copy(col_hbm.at[b], col_v)
        pltpu.sync_copy(row_hbm.at[b], row_v)
        pltpu.sync_copy(y2_hbm, y2_v)

        def step2(j, carry):
            for k in range(CH // L):
                c16 = col_v[j, pl.ds(k * L, L)]
                r16 = row_v[j, pl.ds(k * L, L)]
                for k4 in range(4):
                    vals = plsc.load_gather(y2_v, [c16 + k4])
                    plsc.addupdate_scatter(acc2_v, [r16 + k4 * n_acc], vals)
            return carry

        lax.fori_loop(0, nch, step2, 0)
        pltpu.sync_copy(acc2_v, p2_hbm.at[b])

    return sc_narrow(y2f, col3, row3)


def _combine_body(h_ref, xpt_ref, p1_ref, p2_ref, hn_ref, xnt_ref):
    s1 = p1_ref[0] + p1_ref[1]
    hn_ref[...] = h_ref[...] + s1

    @pl.when(pl.program_id(0) == 0)
    def _():
        s2t = jnp.sum(p2_ref[...], axis=0)
        wsum = s2t[3:4, :]
        krow = lax.broadcasted_iota(jnp.int32, s2t.shape, 0)
        xw = jnp.where(krow < 3, s2t, 0.0)
        xptb = xpt_ref[...]
        xnt_ref[...] = xptb + xptb * wsum - xw


def _combine_stage(h, xpt, p1, p2, bn):
    n, d = h.shape
    na = xpt.shape[1]
    grid = (n // bn,)
    return pl.pallas_call(
        _combine_body,
        grid=grid,
        in_specs=[
            pl.BlockSpec((bn, d), lambda i: (i, 0)),
            pl.BlockSpec((4, na), lambda i: (0, 0)),
            pl.BlockSpec((2, bn, d), lambda i: (0, i, 0)),
            pl.BlockSpec((NW, 4, na), lambda i: (0, 0, 0)),
        ],
        out_specs=[
            pl.BlockSpec((bn, d), lambda i: (i, 0)),
            pl.BlockSpec((4, na), lambda i: (0, 0)),
        ],
        out_shape=[
            jax.ShapeDtypeStruct((n, d), jnp.float32),
            jax.ShapeDtypeStruct((4, na), jnp.float32),
        ],
    )(h, xpt, p1, p2)


def kernel(h, x, edge_index, W_h, b_h, W_x, b_x):
    n, d = h.shape
    e = edge_index.shape[1]
    nch = -(-e // (NW * CH))
    e_pad = NW * nch * CH
    n_acc = (-(-(n + 1) // (NS * 8)) * 8) * NS

    row = edge_index[0].astype(jnp.int32)
    col = edge_index[1].astype(jnp.int32)
    colp = jnp.concatenate([col, jnp.zeros((e_pad - e,), jnp.int32)])
    rowp = jnp.concatenate([row, jnp.full((e_pad - e,), n, jnp.int32)])
    col3 = colp.reshape(NW, nch, CH)
    row3 = rowp.reshape(NW, nch, CH)
    xp = jnp.concatenate([x, jnp.zeros((n, 1), jnp.float32)], axis=1)

    wx4 = jnp.broadcast_to(W_x, (4, d))
    y1, y2 = _node_stage(h, xp, W_h, b_h.reshape(1, d), wx4,
                         b_x.reshape(1, 1), bn=1000)
    nq = e_pad // CH
    p2 = _sc_narrow_stage(y2.reshape(-1), (col3 * 4), row3, n_acc, nch, n)
    p1 = _sc_wide_stage(y1, colp.reshape(nq, CH), rowp.reshape(nq, CH),
                        n_acc, nq)
    xpt = jnp.concatenate(
        [x.T, jnp.zeros((1, n), jnp.float32)], axis=0)
    xpt = jnp.concatenate(
        [xpt, jnp.zeros((4, n_acc - n), jnp.float32)], axis=1)
    h_new, xnt = _combine_stage(h, xpt, p1, p2.reshape(NW, 4, n_acc), bn=1000)
    return (h_new, xnt[:3, :n].T)
